# baseline (speedup 1.0000x reference)
import jax
import jax.numpy as jnp
from jax import lax
from jax.experimental import pallas as pl
from jax.experimental.pallas import tpu as pltpu
from jax.experimental.pallas import tpu_sc as plsc

VOCAB = 1000000
EMBED = 32
ROWS = 4096
COLS = 200
TOTAL = ROWS * COLS

NC = 2
NS = 16
NW = NC * NS
BPW = ROWS // NW
PER_W = BPW * COLS
G = 4
NG = COLS // G


def _embed_body(idx_hbm, table_hbm, out_hbm, idx_v, tk_v, gb_v, tb_v, gsem, ssem):
    c = lax.axis_index("c")
    s = lax.axis_index("s")
    wid = s * NC + c
    ibase = wid * PER_W

    pltpu.sync_copy(idx_hbm.at[pl.ds(ibase, PER_W)], idx_v)

    iota = lax.iota(jnp.int32, 16)
    vstride = iota * COLS
    iota32 = iota * EMBED

    def compact(g, b):
        @plsc.parallel_loop(0, G * (BPW // 16), unroll=4)
        def _(m):
            j = m >> 3
            blk = m & 7
            av = vstride + (blk * 16 * COLS) + (g * G + j)
            tk_v[b, pl.ds(j * BPW + blk * 16, 16)] = plsc.load_gather(idx_v, [av])

    def gather_desc(b):
        return pltpu.make_async_copy(table_hbm.at[tk_v.at[b]], gb_v.at[b], gsem.at[b])

    rbase = iota >> 3
    cbase = (iota & 7) * 128

    def transpose(b):
        @plsc.parallel_loop(0, G * BPW, unroll=4)
        def _(k):
            j = k >> 7
            kk = k & 127
            x0 = gb_v[b, k, pl.ds(0, 16)]
            x1 = gb_v[b, k, pl.ds(16, 16)]
            rowv = rbase + (j * 4)
            colv = cbase + kk
            plsc.store_scatter(tb_v.at[b], [rowv, colv], x0)
            plsc.store_scatter(tb_v.at[b], [rowv + 2, colv], x1)

    def store_desc(g, b):
        return pltpu.make_async_copy(
            tb_v.at[b],
            out_hbm.at[pl.ds(g * G * 4, G * 4), pl.ds(wid * 1024, 1024)],
            ssem.at[b],
        )

    compact(0, 0)
    gather_desc(0).start()
    compact(1, 1)
    gather_desc(1).start()
    gather_desc(0).wait()
    transpose(0)
    store_desc(0, 0).start()
    compact(2, 0)
    gather_desc(0).start()
    gather_desc(1).wait()
    transpose(1)
    store_desc(1, 1).start()

    def steady(o, carry):
        g0 = 2 * o
        compact(g0 + 1, 1)
        gather_desc(1).start()
        gather_desc(0).wait()
        transpose(0)
        store_desc(g0 - 2, 0).wait()
        store_desc(g0, 0).start()
        compact(g0 + 2, 0)
        gather_desc(0).start()
        gather_desc(1).wait()
        transpose(1)
        store_desc(g0 - 1, 1).wait()
        store_desc(g0 + 1, 1).start()
        return carry

    lax.fori_loop(1, NG // 2 - 1, steady, 0)

    compact(NG - 1, 1)
    gather_desc(1).start()
    gather_desc(0).wait()
    transpose(0)
    store_desc(NG - 4, 0).wait()
    store_desc(NG - 2, 0).start()
    gather_desc(1).wait()
    transpose(1)
    store_desc(NG - 3, 1).wait()
    store_desc(NG - 1, 1).start()
    store_desc(NG - 2, 0).wait()
    store_desc(NG - 1, 1).wait()


@jax.jit
def _embed(idx_flat, weight):
    mesh = plsc.VectorSubcoreMesh(core_axis_name="c", subcore_axis_name="s")
    return pl.kernel(
        _embed_body,
        mesh=mesh,
        out_type=jax.ShapeDtypeStruct((COLS * 4, NW * 1024), jnp.float32),
        scratch_types=[
            pltpu.VMEM((PER_W,), jnp.int32),
            pltpu.VMEM((2, G * BPW), jnp.int32),
            pltpu.VMEM((2, G * BPW, EMBED), jnp.float32),
            pltpu.VMEM((2, G * 4, 1024), jnp.float32),
            pltpu.SemaphoreType.DMA((2,)),
            pltpu.SemaphoreType.DMA((2,)),
        ],
        compiler_params=pltpu.CompilerParams(
            use_tc_tiling_on_sc=False, needs_layout_passes=False
        ),
    )(idx_flat, weight)


def kernel(seq, weight):
    idx_flat = seq.reshape(TOTAL).astype(jnp.int32)
    out2 = _embed(idx_flat, weight)
    return (
        out2.reshape(COLS, 4, 32, 8, 128)
        .transpose(2, 4, 0, 1, 3)
        .reshape(ROWS, COLS, EMBED)
    )


# speedup vs baseline: 1.4981x; 1.4981x over previous
import jax
import jax.numpy as jnp
from jax import lax
from jax.experimental import pallas as pl
from jax.experimental.pallas import tpu as pltpu
from jax.experimental.pallas import tpu_sc as plsc

VOCAB = 1000000
EMBED = 32
ROWS = 4096
COLS = 200
TOTAL = ROWS * COLS

NC = 2
NS = 16
NW = NC * NS
BPW = ROWS // NW
PER_W = BPW * COLS
G = 4
NG = COLS // G


def _embed_body(idx_hbm, table_hbm, out_hbm, idx_v, tk_v, gb_v, tb_v, gsem, ssem):
    c = lax.axis_index("c")
    s = lax.axis_index("s")
    wid = s * NC + c
    ibase = wid * PER_W

    pltpu.sync_copy(idx_hbm.at[pl.ds(ibase, PER_W)], idx_v)

    iota = lax.iota(jnp.int32, 16)
    vstride = iota * COLS
    iota32 = iota * EMBED

    def compact(g, b):
        @plsc.parallel_loop(0, G * (BPW // 16), unroll=4)
        def _(m):
            j = m >> 3
            blk = m & 7
            av = vstride + (blk * 16 * COLS) + (g * G + j)
            tk_v[b, pl.ds(j * BPW + blk * 16, 16)] = plsc.load_gather(idx_v, [av])

    def gather_desc(b):
        return pltpu.make_async_copy(table_hbm.at[tk_v.at[b]], gb_v.at[b], gsem.at[b])

    rbase = iota >> 3
    midv = iota & 7

    def transpose(b):
        @plsc.parallel_loop(0, G * BPW, unroll=4)
        def _(k):
            j = k >> 7
            kk = k & 127
            x0 = gb_v[b, k, pl.ds(0, 16)]
            x1 = gb_v[b, k, pl.ds(16, 16)]
            rowv = rbase + (j * 4)
            colv = jnp.full((16,), 0, dtype=jnp.int32) + kk
            plsc.store_scatter(tb_v.at[b], [rowv, midv, colv], x0)
            plsc.store_scatter(tb_v.at[b], [rowv + 2, midv, colv], x1)

    def store_desc(g, b):
        return pltpu.make_async_copy(
            tb_v.at[b, :, :, pl.ds(0, 128)],
            out_hbm.at[pl.ds(g * G * 4, G * 4), pl.ds(wid * 8, 8), :],
            ssem.at[b],
        )

    compact(0, 0)
    gather_desc(0).start()
    compact(1, 1)
    gather_desc(1).start()
    gather_desc(0).wait()
    transpose(0)
    store_desc(0, 0).start()
    compact(2, 0)
    gather_desc(0).start()
    gather_desc(1).wait()
    transpose(1)
    store_desc(1, 1).start()

    def steady(o, carry):
        g0 = 2 * o
        compact(g0 + 1, 1)
        gather_desc(1).start()
        gather_desc(0).wait()
        transpose(0)
        store_desc(g0 - 2, 0).wait()
        store_desc(g0, 0).start()
        compact(g0 + 2, 0)
        gather_desc(0).start()
        gather_desc(1).wait()
        transpose(1)
        store_desc(g0 - 1, 1).wait()
        store_desc(g0 + 1, 1).start()
        return carry

    lax.fori_loop(1, NG // 2 - 1, steady, 0)

    compact(NG - 1, 1)
    gather_desc(1).start()
    gather_desc(0).wait()
    transpose(0)
    store_desc(NG - 4, 0).wait()
    store_desc(NG - 2, 0).start()
    gather_desc(1).wait()
    transpose(1)
    store_desc(NG - 3, 1).wait()
    store_desc(NG - 1, 1).start()
    store_desc(NG - 2, 0).wait()
    store_desc(NG - 1, 1).wait()


@jax.jit
def _embed(idx_flat, weight):
    mesh = plsc.VectorSubcoreMesh(core_axis_name="c", subcore_axis_name="s")
    return pl.kernel(
        _embed_body,
        mesh=mesh,
        out_type=jax.ShapeDtypeStruct((COLS * 4, NW * 8, 128), jnp.float32),
        scratch_types=[
            pltpu.VMEM((PER_W,), jnp.int32),
            pltpu.VMEM((2, G * BPW), jnp.int32),
            pltpu.VMEM((2, G * BPW, EMBED), jnp.float32),
            pltpu.VMEM((2, G * 4, 8, 129), jnp.float32),
            pltpu.SemaphoreType.DMA((2,)),
            pltpu.SemaphoreType.DMA((2,)),
        ],
        compiler_params=pltpu.CompilerParams(
            use_tc_tiling_on_sc=False, needs_layout_passes=False
        ),
    )(idx_flat, weight)


def kernel(seq, weight):
    idx_flat = seq.reshape(TOTAL).astype(jnp.int32)
    out2 = _embed(idx_flat, weight)
    return (
        out2.reshape(COLS, 4, 32, 8, 128)
        .transpose(2, 4, 0, 1, 3)
        .reshape(ROWS, COLS, EMBED)
    )
